# baseline (speedup 1.0000x reference)
import jax
import jax.numpy as jnp
from jax.experimental import pallas as pl
from jax.experimental.pallas import tpu as pltpu

_B = 128
_D = 64
_BLOCK = 4096


def _flash_kernel(x_ref, k_ref, v_ref, o_ref, m_ref, l_ref, acc_ref):
    i = pl.program_id(0)
    n = pl.num_programs(0)

    @pl.when(i == 0)
    def _init():
        m_ref[...] = jnp.full_like(m_ref, -jnp.inf)
        l_ref[...] = jnp.zeros_like(l_ref)
        acc_ref[...] = jnp.zeros_like(acc_ref)

    x = x_ref[...]
    kb = k_ref[...]
    vb = v_ref[...]

    s_a = jax.lax.dot_general(
        x, kb[:, :_D], (((1,), (1,)), ((), ())),
        preferred_element_type=jnp.float32)
    s_b = jax.lax.dot_general(
        x, kb[:, _D:], (((1,), (1,)), ((), ())),
        preferred_element_type=jnp.float32)

    m_prev = m_ref[...]
    m_cur = jnp.maximum(jnp.max(s_a, axis=1, keepdims=True),
                        jnp.max(s_b, axis=1, keepdims=True))
    m_new = jnp.maximum(m_prev, m_cur)

    alpha = jnp.exp(m_prev - m_new)
    p_a = jnp.exp(s_a - m_new[:, 0:1])
    p_b = jnp.exp(s_b - m_new[:, 0:1])

    l_cur = (jnp.sum(p_a, axis=1, keepdims=True)
             + jnp.sum(p_b, axis=1, keepdims=True))
    l_ref[...] = l_ref[...] * alpha + l_cur
    m_ref[...] = m_new

    pv = (jax.lax.dot_general(
              p_a, vb[:, :_D], (((1,), (0,)), ((), ())),
              preferred_element_type=jnp.float32)
          + jax.lax.dot_general(
              p_b, vb[:, _D:], (((1,), (0,)), ((), ())),
              preferred_element_type=jnp.float32))
    acc_ref[...] = acc_ref[...] * alpha[:, 0:1] + pv

    @pl.when(i == n - 1)
    def _finish():
        o_ref[...] = acc_ref[...] / l_ref[...][:, 0:1]


def kernel(x, memory_keys, memory_values):
    m_total = memory_keys.shape[0]
    k2 = memory_keys.reshape(m_total // 2, 2 * _D)
    v2 = memory_values.reshape(m_total // 2, 2 * _D)
    grid = ((m_total // 2) // _BLOCK,)
    return pl.pallas_call(
        _flash_kernel,
        grid=grid,
        in_specs=[
            pl.BlockSpec((_B, _D), lambda i: (0, 0)),
            pl.BlockSpec((_BLOCK, 2 * _D), lambda i: (i, 0)),
            pl.BlockSpec((_BLOCK, 2 * _D), lambda i: (i, 0)),
        ],
        out_specs=pl.BlockSpec((_B, _D), lambda i: (0, 0)),
        out_shape=jax.ShapeDtypeStruct((_B, _D), jnp.float32),
        scratch_shapes=[
            pltpu.VMEM((_B, 128), jnp.float32),
            pltpu.VMEM((_B, 128), jnp.float32),
            pltpu.VMEM((_B, _D), jnp.float32),
        ],
        compiler_params=pltpu.CompilerParams(
            dimension_semantics=("arbitrary",),
        ),
    )(x, k2, v2)


# speedup vs baseline: 1.3390x; 1.3390x over previous
import jax
import jax.numpy as jnp
from jax.experimental import pallas as pl
from jax.experimental.pallas import tpu as pltpu

_B = 128
_D = 64
_BLOCK_M = 8192


def _flash_kernel(x_ref, k_ref, v_ref, o_ref, m_ref, l_ref, acc_ref):
    i = pl.program_id(0)
    n = pl.num_programs(0)

    @pl.when(i == 0)
    def _init():
        m_ref[...] = jnp.full_like(m_ref, -jnp.inf)
        l_ref[...] = jnp.zeros_like(l_ref)
        acc_ref[...] = jnp.zeros_like(acc_ref)

    x = x_ref[...]
    k = k_ref[...]
    s = jax.lax.dot_general(
        x, k, (((1,), (1,)), ((), ())),
        preferred_element_type=jnp.float32)

    m_prev = m_ref[...]
    m_cur = jnp.max(s, axis=1, keepdims=True)
    m_new = jnp.maximum(m_prev, m_cur)

    alpha = jnp.exp(m_prev - m_new)
    p = jnp.exp(s - m_new[:, 0:1])

    l_cur = jnp.sum(p, axis=1, keepdims=True)
    l_ref[...] = l_ref[...] * alpha + l_cur
    m_ref[...] = m_new

    pv = jax.lax.dot_general(
        p, v_ref[...], (((1,), (0,)), ((), ())),
        preferred_element_type=jnp.float32)
    acc_ref[...] = acc_ref[...] * alpha[:, 0:1] + pv

    @pl.when(i == n - 1)
    def _finish():
        o_ref[...] = acc_ref[...] / l_ref[...][:, 0:1]


def kernel(x, memory_keys, memory_values):
    m_total = memory_keys.shape[0]
    grid = (m_total // _BLOCK_M,)
    return pl.pallas_call(
        _flash_kernel,
        grid=grid,
        in_specs=[
            pl.BlockSpec((_B, _D), lambda i: (0, 0)),
            pl.BlockSpec((_BLOCK_M, _D), lambda i: (i, 0)),
            pl.BlockSpec((_BLOCK_M, _D), lambda i: (i, 0)),
        ],
        out_specs=pl.BlockSpec((_B, _D), lambda i: (0, 0)),
        out_shape=jax.ShapeDtypeStruct((_B, _D), jnp.float32),
        scratch_shapes=[
            pltpu.VMEM((_B, 128), jnp.float32),
            pltpu.VMEM((_B, 128), jnp.float32),
            pltpu.VMEM((_B, _D), jnp.float32),
        ],
        compiler_params=pltpu.CompilerParams(
            dimension_semantics=("arbitrary",),
        ),
    )(x, memory_keys, memory_values)


# speedup vs baseline: 1.3771x; 1.0284x over previous
import jax
import jax.numpy as jnp
from jax.experimental import pallas as pl
from jax.experimental.pallas import tpu as pltpu

_B = 128
_D = 64
_BLOCK_M = 8192


def _flash_kernel(x_ref, k1_ref, k2_ref, v1_ref, v2_ref, o_ref,
                  m_ref, l_ref, acc_ref):
    i = pl.program_id(0)
    n = pl.num_programs(0)

    @pl.when(i == 0)
    def _init():
        m_ref[...] = jnp.full_like(m_ref, -jnp.inf)
        l_ref[...] = jnp.zeros_like(l_ref)
        acc_ref[...] = jnp.zeros_like(acc_ref)

    x = x_ref[...]
    s1 = jax.lax.dot_general(
        x, k1_ref[...], (((1,), (1,)), ((), ())),
        preferred_element_type=jnp.float32)
    s2 = jax.lax.dot_general(
        x, k2_ref[...], (((1,), (1,)), ((), ())),
        preferred_element_type=jnp.float32)

    m_prev = m_ref[...]
    m_cur = jnp.maximum(jnp.max(s1, axis=1, keepdims=True),
                        jnp.max(s2, axis=1, keepdims=True))
    m_new = jnp.maximum(m_prev, m_cur)

    alpha = jnp.exp(m_prev - m_new)
    p1 = jnp.exp(s1 - m_new[:, 0:1])
    p2 = jnp.exp(s2 - m_new[:, 0:1])

    l_cur = (jnp.sum(p1, axis=1, keepdims=True)
             + jnp.sum(p2, axis=1, keepdims=True))
    l_ref[...] = l_ref[...] * alpha + l_cur
    m_ref[...] = m_new

    pv = (jax.lax.dot_general(
              p1, v1_ref[...], (((1,), (0,)), ((), ())),
              preferred_element_type=jnp.float32)
          + jax.lax.dot_general(
              p2, v2_ref[...], (((1,), (0,)), ((), ())),
              preferred_element_type=jnp.float32))
    acc_ref[...] = acc_ref[...] * alpha[:, 0:1] + pv

    @pl.when(i == n - 1)
    def _finish():
        o_ref[...] = acc_ref[...] / l_ref[...][:, 0:1]


def kernel(x, memory_keys, memory_values):
    m_total = memory_keys.shape[0]
    n = (m_total // _BLOCK_M) // 2
    return pl.pallas_call(
        _flash_kernel,
        grid=(n,),
        in_specs=[
            pl.BlockSpec((_B, _D), lambda i: (0, 0)),
            pl.BlockSpec((_BLOCK_M, _D), lambda i: (i, 0)),
            pl.BlockSpec((_BLOCK_M, _D), lambda i, _n=n: (i + _n, 0)),
            pl.BlockSpec((_BLOCK_M, _D), lambda i: (i, 0)),
            pl.BlockSpec((_BLOCK_M, _D), lambda i, _n=n: (i + _n, 0)),
        ],
        out_specs=pl.BlockSpec((_B, _D), lambda i: (0, 0)),
        out_shape=jax.ShapeDtypeStruct((_B, _D), jnp.float32),
        scratch_shapes=[
            pltpu.VMEM((_B, 128), jnp.float32),
            pltpu.VMEM((_B, 128), jnp.float32),
            pltpu.VMEM((_B, _D), jnp.float32),
        ],
        compiler_params=pltpu.CompilerParams(
            dimension_semantics=("arbitrary",),
        ),
    )(x, memory_keys, memory_keys, memory_values, memory_values)
